# Initial kernel scaffold; baseline (speedup 1.0000x reference)
import jax
import jax.numpy as jnp
from jax.experimental import pallas as pl


def kernel(quant_pred, target_wav, ze, emb, min_dist, gamma):
    raise NotImplementedError("write your pallas kernel here")


# speedup vs baseline: 2.6748x; 2.6748x over previous
import functools

import jax
import jax.numpy as jnp
from jax.experimental import pallas as pl

B, C, N, Q, K = 8, 256, 2048, 64, 1024
NB = 512


def _body(qp_ref, tgt_ref, ze_ref, emb_ref, md_ref, out_ref):
    i = pl.program_id(0)

    emb_v = emb_ref[...]
    emb_sq = jnp.sum(emb_v * emb_v, axis=1)
    ze_v = ze_ref[...]
    ze_sq = jnp.sum(ze_v * ze_v, axis=1)

    acc = jnp.float32(0.0)
    for b in range(B):
        cross = jnp.dot(emb_v, ze_v[b],
                        preferred_element_type=jnp.float32)
        d = emb_sq[:, None] - 2.0 * cross
        acc += jnp.sum(jnp.min(d, axis=0))
    acc += jnp.sum(ze_sq)

    x = qp_ref[...]
    mx = jnp.max(x, axis=1)
    lse = jnp.log(jnp.sum(jnp.exp(x - mx[:, None, :]), axis=1)) + mx
    cidx = jax.lax.broadcasted_iota(jnp.int32, x.shape, 1)
    tv = jnp.sum(jnp.where(cidx == tgt_ref[...][:, None, :], x, 0.0), axis=1)
    acc += jnp.sum(lse - tv)

    md_sum = jnp.sum(md_ref[...])

    @pl.when(i == 0)
    def _():
        out_ref[...] = jnp.zeros_like(out_ref)

    out_ref[0, :] += jnp.broadcast_to(acc, (128,))
    out_ref[1, :] += jnp.broadcast_to(md_sum, (128,))


def kernel(quant_pred, target_wav, ze, emb, min_dist, gamma=0.25):
    tgt = target_wav.astype(jnp.int32)
    out = pl.pallas_call(
        _body,
        grid=(N // NB,),
        in_specs=[
            pl.BlockSpec((B, C, NB), lambda i: (0, 0, i)),
            pl.BlockSpec((B, NB), lambda i: (0, i)),
            pl.BlockSpec((B, Q, NB), lambda i: (0, 0, i)),
            pl.BlockSpec((K, Q), lambda i: (0, 0)),
            pl.BlockSpec((B, NB), lambda i: (0, i)),
        ],
        out_specs=pl.BlockSpec((2, 128), lambda i: (0, 0)),
        out_shape=jax.ShapeDtypeStruct((2, 128), jnp.float32),
    )(quant_pred, tgt, ze, emb, min_dist)
    return out[0, 0] + gamma * out[1, 0]
